# Initial kernel scaffold; baseline (speedup 1.0000x reference)
import jax
import jax.numpy as jnp
from jax.experimental import pallas as pl


def kernel(tokens, table, avg, var):
    raise NotImplementedError("write your pallas kernel here")


# speedup vs baseline: 4.1185x; 4.1185x over previous
import functools

import jax
import jax.numpy as jnp
from jax import lax
from jax.experimental import pallas as pl
from jax.experimental.pallas import tpu as pltpu
from jax.experimental.pallas import tpu_sc as plsc

_VOCAB = 100000
_D = 64
_NT = 5
_SEQ = 200
_NG = _SEQ - _NT
_IDXP = 208
_L = 16


def _build(B, NC, NS):
    NW = NC * NS
    bpw = B // NW
    mesh = plsc.VectorSubcoreMesh(core_axis_name="c", subcore_axis_name="s")

    @functools.partial(
        pl.kernel,
        out_type=jax.ShapeDtypeStruct((B * _SEQ, _D), jnp.float32),
        mesh=mesh,
        compiler_params=pltpu.CompilerParams(use_tc_tiling_on_sc=False),
        scratch_types=[
            pltpu.VMEM((_IDXP,), jnp.int32),
            pltpu.VMEM((_SEQ, _D), jnp.float32),
            pltpu.VMEM((_NT * _D,), jnp.float32),
            pltpu.VMEM((_NT * _D,), jnp.float32),
            pltpu.VMEM((_NT * _D,), jnp.float32),
            pltpu.SemaphoreType.DMA,
        ],
    )
    def k(idx_hbm, table_hbm, samp_hbm, var_hbm, avg_hbm, out_hbm,
          idx_v, buf_v, samp_v, var_v, avg_v, sem):
        wid = lax.axis_index("s") * NC + lax.axis_index("c")
        pltpu.sync_copy(var_hbm, var_v)
        pltpu.sync_copy(avg_hbm, avg_v)

        def body(i, carry):
            b = wid * bpw + i
            pltpu.sync_copy(idx_hbm.at[b], idx_v)
            pltpu.sync_copy(samp_hbm.at[b], samp_v)
            g1 = pltpu.async_copy(
                table_hbm.at[idx_v.at[pl.ds(0, 128)]],
                buf_v.at[pl.ds(0, 128)], sem)
            g2 = pltpu.async_copy(
                table_hbm.at[idx_v.at[pl.ds(128, _NG - 128)]],
                buf_v.at[pl.ds(128, _NG - 128)], sem)
            for j in range(_NT * _D // _L):
                r, c = divmod(j, _D // _L)
                sl = pl.ds(j * _L, _L)
                buf_v[_NG + r, pl.ds((c * _L), _L)] = (
                    samp_v[sl] * var_v[sl] + avg_v[sl])
            g1.wait()
            g2.wait()
            pltpu.sync_copy(buf_v, out_hbm.at[pl.ds(b * _SEQ, _SEQ)])
            return carry

        lax.fori_loop(0, bpw, body, 0)

    return k


def kernel(tokens, table, avg, var):
    B = tokens.shape[0]
    idx = jnp.pad(tokens[:, _NT:], ((0, 0), (0, _IDXP - _NG)))
    sample = jax.random.normal(jax.random.key(1), (B, _NT, _D),
                               dtype=jnp.float32)
    info = plsc.get_sparse_core_info()
    k = _build(B, info.num_cores, info.num_subcores)
    out = k(idx, table, sample.reshape(B, _NT * _D),
            var.reshape(_NT * _D), avg.reshape(_NT * _D))
    return out.reshape(B, _SEQ, _D)
